# Initial kernel scaffold; baseline (speedup 1.0000x reference)
import functools

import jax
import jax.numpy as jnp
from jax import lax
from jax.experimental import pallas as pl
from jax.experimental.pallas import tpu as pltpu
from jax.experimental.pallas import tpu_sc as plsc

N = 100000
E = 1600000
D = 32

NC = 2
NS = 16

N_BUF = 100352
H = N_BUF // 2
ACC_ROWS = H + 16

ER = 12544
E_PAD = ER * 128
PAD_IDX = N

TROWS = ER // NS
DEG_SC = 16
DEG_NCH = TROWS // DEG_SC
RED_W = N_BUF // NS

PROP_SCR = 8
PROP_NCH = TROWS // PROP_SCR
EPI_R = 448

_mesh = plsc.VectorSubcoreMesh(core_axis_name="c", subcore_axis_name="s")


@functools.partial(
    pl.kernel,
    out_type=jax.ShapeDtypeStruct((2, N_BUF), jnp.float32),
    mesh=_mesh,
    scratch_types=[
        pltpu.VMEM_SHARED((N_BUF,), jnp.float32),
        pltpu.VMEM((DEG_SC, 128), jnp.int32),
        pltpu.VMEM((128,), jnp.float32),
        pltpu.VMEM((RED_W,), jnp.float32),
    ],
)
def _deg_kernel(ht_hbm, deg_out, hist_sh, idxb, onesb, stage):
    cid = lax.axis_index("c")
    sid = lax.axis_index("s")

    zero16 = jnp.zeros((16,), jnp.float32)

    def _z(i, _):
        stage[pl.ds(i * 16, 16)] = zero16
        return 0

    lax.fori_loop(0, RED_W // 16, _z, 0)
    for j in range(8):
        onesb[pl.ds(j * 16, 16)] = jnp.ones((16,), jnp.float32)
    pltpu.sync_copy(stage, hist_sh.at[pl.ds(sid * RED_W, RED_W)])
    plsc.subcore_barrier()

    def _chunk(c, _):
        rb = sid * TROWS + c * DEG_SC
        pltpu.sync_copy(ht_hbm.at[cid, pl.ds(rb, DEG_SC)], idxb)
        for j in range(DEG_SC):
            pltpu.sync_copy(onesb, hist_sh.at[idxb.at[j]], add=True)
        return 0

    lax.fori_loop(0, DEG_NCH, _chunk, 0)
    plsc.subcore_barrier()

    pltpu.sync_copy(hist_sh.at[pl.ds(sid * RED_W, RED_W)], stage)
    pltpu.sync_copy(stage, deg_out.at[cid, pl.ds(sid * RED_W, RED_W)])


@functools.partial(
    pl.kernel,
    out_type=jax.ShapeDtypeStruct((N_BUF, D), jnp.float32),
    mesh=_mesh,
    scratch_types=[
        pltpu.VMEM_SHARED((ACC_ROWS, D), jnp.float32),
        pltpu.VMEM((PROP_SCR, 128), jnp.int32),
        pltpu.VMEM((PROP_SCR, 128), jnp.int32),
        pltpu.VMEM((PROP_SCR, 128), jnp.int32),
        pltpu.VMEM((PROP_SCR * 128, D), jnp.float32),
        pltpu.VMEM((EPI_R, D), jnp.float32),
        pltpu.SemaphoreType.DMA,
    ],
)
def _prop_kernel(u_hbm, h_hbm, t_hbm, z_hbm, out_hbm,
                 acc_sh, hb, tb, ib, rows, stage, sem):
    cid = lax.axis_index("c")
    sid = lax.axis_index("s")
    base = cid * H
    r0 = sid * (H // NS)

    pltpu.sync_copy(z_hbm, rows)
    for k in range(3):
        pltpu.sync_copy(rows, acc_sh.at[pl.ds(r0 + k * 1024, 1024)])
    pltpu.sync_copy(rows.at[pl.ds(0, 64)], acc_sh.at[pl.ds(r0 + 3072, 64)])
    plsc.subcore_barrier()

    def _chunk(c, _):
        rb = sid * TROWS + c * PROP_SCR
        pltpu.sync_copy(h_hbm.at[pl.ds(rb, PROP_SCR)], hb)
        pltpu.sync_copy(t_hbm.at[pl.ds(rb, PROP_SCR)], tb)
        descs = []
        for j in range(PROP_SCR):
            descs.append(pltpu.async_copy(
                u_hbm.at[tb.at[j]], rows.at[pl.ds(j * 128, 128)], sem))
        for j in range(PROP_SCR):
            def _grp(k, _, j=j):
                hv = hb[j, pl.ds(k * 16, 16)]
                ih = hv - base
                ok = (ih >= 0) & (ih < H)
                ib[j, pl.ds(k * 16, 16)] = jnp.where(ok, ih, H)
                return 0
            lax.fori_loop(0, 8, _grp, 0)
        for d in descs:
            d.wait()
        for j in range(PROP_SCR):
            pltpu.sync_copy(rows.at[pl.ds(j * 128, 128)],
                            acc_sh.at[ib.at[j]], add=True)
        return 0

    lax.fori_loop(0, PROP_NCH, _chunk, 0)
    plsc.subcore_barrier()

    for c in range(7):
        pltpu.sync_copy(acc_sh.at[pl.ds(r0 + c * EPI_R, EPI_R)], stage)
        pltpu.sync_copy(stage, out_hbm.at[pl.ds(base + r0 + c * EPI_R, EPI_R)])


_TB = 2048
_TGRID = N_BUF // _TB


def _t1_body(dh_ref, dt_ref, x_ref, u1_ref, rh_ref, rhrt_ref):
    rh = lax.rsqrt(jnp.maximum(dh_ref[...], 1.0))
    rt = lax.rsqrt(jnp.maximum(dt_ref[...], 1.0))
    rhb = jnp.broadcast_to(rh, (_TB, D))
    rtb = jnp.broadcast_to(rt, (_TB, D))
    u1_ref[...] = x_ref[...] * rtb
    rh_ref[...] = rhb
    rhrt_ref[...] = rhb * rtb


_t1 = pl.pallas_call(
    _t1_body,
    out_shape=(jax.ShapeDtypeStruct((N_BUF, D), jnp.float32),) * 3,
    grid=(_TGRID,),
    in_specs=[
        pl.BlockSpec((_TB, 1), lambda i: (i, 0)),
        pl.BlockSpec((_TB, 1), lambda i: (i, 0)),
        pl.BlockSpec((_TB, D), lambda i: (i, 0)),
    ],
    out_specs=(pl.BlockSpec((_TB, D), lambda i: (i, 0)),) * 3,
)


def _t2_body(x_ref, raw_ref, rh_ref, rhrt_ref, acc_ref, u2_ref):
    raw = raw_ref[...]
    acc_ref[...] = x_ref[...] + raw * rh_ref[...]
    u2_ref[...] = raw * rhrt_ref[...]


_t2 = pl.pallas_call(
    _t2_body,
    out_shape=(jax.ShapeDtypeStruct((N_BUF, D), jnp.float32),) * 2,
    grid=(_TGRID,),
    in_specs=[pl.BlockSpec((_TB, D), lambda i: (i, 0))] * 4,
    out_specs=(pl.BlockSpec((_TB, D), lambda i: (i, 0)),) * 2,
)


def _t3_body(acc_ref, raw_ref, rh_ref, out_ref):
    out_ref[...] = (acc_ref[...] + raw_ref[...] * rh_ref[...]) * (1.0 / 3.0)


_t3 = pl.pallas_call(
    _t3_body,
    out_shape=jax.ShapeDtypeStruct((N_BUF, D), jnp.float32),
    grid=(_TGRID,),
    in_specs=[pl.BlockSpec((_TB, D), lambda i: (i, 0))] * 3,
    out_specs=pl.BlockSpec((_TB, D), lambda i: (i, 0)),
)


def kernel(x, edge_index):
    pad = jnp.full((E_PAD - E,), PAD_IDX, jnp.int32)
    h2 = jnp.concatenate([edge_index[0], pad]).reshape(ER, 128)
    t2 = jnp.concatenate([edge_index[1], pad]).reshape(ER, 128)
    ht = jnp.stack([h2, t2])
    xp = jnp.pad(x, ((0, N_BUF - N), (0, 0)))
    zeros2 = jnp.zeros((PROP_SCR * 128, D), jnp.float32)

    deg = _deg_kernel(ht)
    degh = deg[0].reshape(N_BUF, 1)
    degt = deg[1].reshape(N_BUF, 1)
    u1, rh_exp, rhrt_exp = _t1(degh, degt, xp)
    raw1 = _prop_kernel(u1, h2, t2, zeros2)
    acc1, u2 = _t2(xp, raw1, rh_exp, rhrt_exp)
    raw2 = _prop_kernel(u2, h2, t2, zeros2)
    out = _t3(acc1, raw2, rh_exp)
    return out[:N]


# speedup vs baseline: 15.8230x; 15.8230x over previous
import functools

import jax
import jax.numpy as jnp
from jax import lax
from jax.experimental import pallas as pl
from jax.experimental.pallas import tpu as pltpu
from jax.experimental.pallas import tpu_sc as plsc

N = 100000
E = 1600000
D = 32

NC = 2
NS = 16

N_BUF = 100352
H = N_BUF // 2
ACC_ROWS = H + 16

ER = 12544
E_PAD = ER * 128
PAD_IDX = N

TROWS = ER // NS
DEG_SC = 16
DEG_NCH = TROWS // DEG_SC
RED_W = N_BUF // NS

PROP_SCR = 4
PROP_NCH = TROWS // PROP_SCR
EPI_R = 112

_mesh = plsc.VectorSubcoreMesh(core_axis_name="c", subcore_axis_name="s")


@functools.partial(
    pl.kernel,
    out_type=jax.ShapeDtypeStruct((2, N_BUF), jnp.float32),
    mesh=_mesh,
    scratch_types=[
        pltpu.VMEM_SHARED((N_BUF,), jnp.float32),
        pltpu.VMEM((DEG_SC, 128), jnp.int32),
        pltpu.VMEM((128,), jnp.float32),
        pltpu.VMEM((RED_W,), jnp.float32),
    ],
    compiler_params=pltpu.CompilerParams(use_tc_tiling_on_sc=False),
)
def _deg_kernel(ht_hbm, deg_out, hist_sh, idxb, onesb, stage):
    cid = lax.axis_index("c")
    sid = lax.axis_index("s")

    zero16 = jnp.zeros((16,), jnp.float32)

    def _z(i, _):
        stage[pl.ds(i * 16, 16)] = zero16
        return 0

    lax.fori_loop(0, RED_W // 16, _z, 0)
    for j in range(8):
        onesb[pl.ds(j * 16, 16)] = jnp.ones((16,), jnp.float32)
    pltpu.sync_copy(stage, hist_sh.at[pl.ds(sid * RED_W, RED_W)])
    plsc.subcore_barrier()

    def _chunk(c, _):
        rb = sid * TROWS + c * DEG_SC
        pltpu.sync_copy(ht_hbm.at[cid, pl.ds(rb, DEG_SC)], idxb)
        for j in range(DEG_SC):
            pltpu.sync_copy(onesb, hist_sh.at[idxb.at[j]], add=True)
        return 0

    lax.fori_loop(0, DEG_NCH, _chunk, 0)
    plsc.subcore_barrier()

    pltpu.sync_copy(hist_sh.at[pl.ds(sid * RED_W, RED_W)], stage)
    pltpu.sync_copy(stage, deg_out.at[cid, pl.ds(sid * RED_W, RED_W)])


@functools.partial(
    pl.kernel,
    out_type=jax.ShapeDtypeStruct((N_BUF, D), jnp.float32),
    mesh=_mesh,
    scratch_types=[
        pltpu.VMEM_SHARED((ACC_ROWS, D), jnp.float32),
        pltpu.VMEM((PROP_SCR, 128), jnp.int32),
        pltpu.VMEM((PROP_SCR, 128), jnp.int32),
        pltpu.VMEM((PROP_SCR, 128), jnp.int32),
        pltpu.VMEM((PROP_SCR * 128, D), jnp.float32),
        pltpu.VMEM((EPI_R, D), jnp.float32),
        pltpu.SemaphoreType.DMA,
    ],
    compiler_params=pltpu.CompilerParams(use_tc_tiling_on_sc=False),
)
def _prop_kernel(u_hbm, h_hbm, t_hbm, z_hbm, out_hbm,
                 acc_sh, hb, tb, ib, rows, stage, sem):
    cid = lax.axis_index("c")
    sid = lax.axis_index("s")
    base = cid * H
    r0 = sid * (H // NS)

    pltpu.sync_copy(z_hbm, rows)
    for k in range(6):
        pltpu.sync_copy(rows, acc_sh.at[pl.ds(r0 + k * 512, 512)])
    pltpu.sync_copy(rows.at[pl.ds(0, 64)], acc_sh.at[pl.ds(r0 + 3072, 64)])
    plsc.subcore_barrier()

    def _chunk(c, _):
        rb = sid * TROWS + c * PROP_SCR
        pltpu.sync_copy(h_hbm.at[pl.ds(rb, PROP_SCR)], hb)
        pltpu.sync_copy(t_hbm.at[pl.ds(rb, PROP_SCR)], tb)
        descs = []
        for j in range(PROP_SCR):
            descs.append(pltpu.async_copy(
                u_hbm.at[tb.at[j]], rows.at[pl.ds(j * 128, 128)], sem))
        for j in range(PROP_SCR):
            def _grp(k, _, j=j):
                hv = hb[j, pl.ds(k * 16, 16)]
                ih = hv - base
                ok = (ih >= 0) & (ih < H)
                ib[j, pl.ds(k * 16, 16)] = jnp.where(ok, ih, H)
                return 0
            lax.fori_loop(0, 8, _grp, 0)
        for d in descs:
            d.wait()
        for j in range(PROP_SCR):
            pltpu.sync_copy(rows.at[pl.ds(j * 128, 128)],
                            acc_sh.at[ib.at[j]], add=True)
        return 0

    lax.fori_loop(0, PROP_NCH, _chunk, 0)
    plsc.subcore_barrier()

    for c in range(28):
        pltpu.sync_copy(acc_sh.at[pl.ds(r0 + c * EPI_R, EPI_R)], stage)
        pltpu.sync_copy(stage, out_hbm.at[pl.ds(base + r0 + c * EPI_R, EPI_R)])


_TB = 2048
_TGRID = N_BUF // _TB


def _t1_body(dh_ref, dt_ref, x_ref, u1_ref, rh_ref, rhrt_ref):
    rh = lax.rsqrt(jnp.maximum(dh_ref[...], 1.0))
    rt = lax.rsqrt(jnp.maximum(dt_ref[...], 1.0))
    rhb = jnp.broadcast_to(rh, (_TB, D))
    rtb = jnp.broadcast_to(rt, (_TB, D))
    u1_ref[...] = x_ref[...] * rtb
    rh_ref[...] = rhb
    rhrt_ref[...] = rhb * rtb


_t1 = pl.pallas_call(
    _t1_body,
    out_shape=(jax.ShapeDtypeStruct((N_BUF, D), jnp.float32),) * 3,
    grid=(_TGRID,),
    in_specs=[
        pl.BlockSpec((_TB, 1), lambda i: (i, 0)),
        pl.BlockSpec((_TB, 1), lambda i: (i, 0)),
        pl.BlockSpec((_TB, D), lambda i: (i, 0)),
    ],
    out_specs=(pl.BlockSpec((_TB, D), lambda i: (i, 0)),) * 3,
)


def _t2_body(x_ref, raw_ref, rh_ref, rhrt_ref, acc_ref, u2_ref):
    raw = raw_ref[...]
    acc_ref[...] = x_ref[...] + raw * rh_ref[...]
    u2_ref[...] = raw * rhrt_ref[...]


_t2 = pl.pallas_call(
    _t2_body,
    out_shape=(jax.ShapeDtypeStruct((N_BUF, D), jnp.float32),) * 2,
    grid=(_TGRID,),
    in_specs=[pl.BlockSpec((_TB, D), lambda i: (i, 0))] * 4,
    out_specs=(pl.BlockSpec((_TB, D), lambda i: (i, 0)),) * 2,
)


def _t3_body(acc_ref, raw_ref, rh_ref, out_ref):
    out_ref[...] = (acc_ref[...] + raw_ref[...] * rh_ref[...]) * (1.0 / 3.0)


_t3 = pl.pallas_call(
    _t3_body,
    out_shape=jax.ShapeDtypeStruct((N_BUF, D), jnp.float32),
    grid=(_TGRID,),
    in_specs=[pl.BlockSpec((_TB, D), lambda i: (i, 0))] * 3,
    out_specs=pl.BlockSpec((_TB, D), lambda i: (i, 0)),
)


def kernel(x, edge_index):
    pad = jnp.full((E_PAD - E,), PAD_IDX, jnp.int32)
    h2 = jnp.concatenate([edge_index[0], pad]).reshape(ER, 128)
    t2 = jnp.concatenate([edge_index[1], pad]).reshape(ER, 128)
    ht = jnp.stack([h2, t2])
    xp = jnp.pad(x, ((0, N_BUF - N), (0, 0)))
    zeros2 = jnp.zeros((PROP_SCR * 128, D), jnp.float32)

    deg = _deg_kernel(ht)
    degh = deg[0].reshape(N_BUF, 1)
    degt = deg[1].reshape(N_BUF, 1)
    u1, rh_exp, rhrt_exp = _t1(degh, degt, xp)
    raw1 = _prop_kernel(u1, h2, t2, zeros2)
    acc1, u2 = _t2(xp, raw1, rh_exp, rhrt_exp)
    raw2 = _prop_kernel(u2, h2, t2, zeros2)
    out = _t3(acc1, raw2, rh_exp)
    return out[:N]
